# baseline (speedup 1.0000x reference)
import functools

import jax
import jax.numpy as jnp
from jax import lax
from jax.experimental import pallas as pl
from jax.experimental.pallas import tpu as pltpu
from jax.experimental.pallas import tpu_sc as plsc

BATCH = 16384
D = 64
PAIR = 2 * D
NC = 2
NS = 16
L = 16
NW = NC * NS
BPW = BATCH // NW
CHUNK = 128
NCHUNK = BPW // CHUNK
GPC = CHUNK // L


def _bpr_body(u_hbm, i_hbm, j_hbm, ue_hbm, ie_hbm, out_hbm,
              idx_u, idx_i, idx_j, pr_u, pr_i, pr_j,
              rows_u, rows_i, rows_j, out_v, sem0, sem1):
    wid = lax.axis_index("s") * NC + lax.axis_index("c")
    base = wid * BPW

    pltpu.sync_copy(u_hbm.at[pl.ds(base, BPW)], idx_u)
    pltpu.sync_copy(i_hbm.at[pl.ds(base, BPW)], idx_i)
    pltpu.sync_copy(j_hbm.at[pl.ds(base, BPW)], idx_j)

    sems = (sem0, sem1)
    lanes = lax.iota(jnp.int32, L)

    def halve_chunk(c):
        def g_body(g, carry):
            o = c * CHUNK + g * L
            pr_u[pl.ds(o, L)] = lax.shift_right_logical(idx_u[pl.ds(o, L)], 1)
            pr_i[pl.ds(o, L)] = lax.shift_right_logical(idx_i[pl.ds(o, L)], 1)
            pr_j[pl.ds(o, L)] = lax.shift_right_logical(idx_j[pl.ds(o, L)], 1)
            return carry
        lax.fori_loop(0, GPC, g_body, 0)

    def fire_chunk(c):
        buf = c % 2
        sl = pl.ds(c * CHUNK, CHUNK)
        return (
            pltpu.async_copy(ue_hbm.at[pr_u.at[sl]], rows_u.at[buf], sems[buf]),
            pltpu.async_copy(ie_hbm.at[pr_i.at[sl]], rows_i.at[buf], sems[buf]),
            pltpu.async_copy(ie_hbm.at[pr_j.at[sl]], rows_j.at[buf], sems[buf]),
        )

    def compute_chunk(c):
        buf = c % 2
        ru, ri, rj = rows_u.at[buf], rows_i.at[buf], rows_j.at[buf]

        def group_body(g, carry):
            o = c * CHUNK + g * L
            rows_in = g * L + lanes
            half_u = (idx_u[pl.ds(o, L)] & 1) * D
            half_i = (idx_i[pl.ds(o, L)] & 1) * D
            half_j = (idx_j[pl.ds(o, L)] & 1) * D
            acc = jnp.zeros((L,), jnp.float32)
            for d in range(D):
                col = (lanes + d) & (D - 1)
                ue = plsc.load_gather(ru, [rows_in, half_u + col])
                ie = plsc.load_gather(ri, [rows_in, half_i + col])
                je = plsc.load_gather(rj, [rows_in, half_j + col])
                acc = acc + ue * (ie - je)
            out_v[pl.ds(o, L)] = acc
            return carry

        lax.fori_loop(0, GPC, group_body, 0)

    halve_chunk(0)
    copies = fire_chunk(0)
    for c in range(NCHUNK):
        if c + 1 < NCHUNK:
            halve_chunk(c + 1)
            next_copies = fire_chunk(c + 1)
        for cp in copies:
            cp.wait()
        compute_chunk(c)
        if c + 1 < NCHUNK:
            copies = next_copies

    pltpu.sync_copy(out_v, out_hbm.at[pl.ds(base, BPW)])


@functools.partial(
    pl.kernel,
    out_type=jax.ShapeDtypeStruct((BATCH,), jnp.float32),
    mesh=plsc.VectorSubcoreMesh(
        core_axis_name="c", subcore_axis_name="s", num_cores=NC, num_subcores=NS
    ),
    scratch_types=[
        pltpu.VMEM((BPW,), jnp.int32),
        pltpu.VMEM((BPW,), jnp.int32),
        pltpu.VMEM((BPW,), jnp.int32),
        pltpu.VMEM((BPW,), jnp.int32),
        pltpu.VMEM((BPW,), jnp.int32),
        pltpu.VMEM((BPW,), jnp.int32),
        pltpu.VMEM((2, CHUNK, PAIR), jnp.float32),
        pltpu.VMEM((2, CHUNK, PAIR), jnp.float32),
        pltpu.VMEM((2, CHUNK, PAIR), jnp.float32),
        pltpu.VMEM((BPW,), jnp.float32),
        pltpu.SemaphoreType.DMA,
        pltpu.SemaphoreType.DMA,
    ],
    compiler_params=pltpu.CompilerParams(
        needs_layout_passes=False, use_tc_tiling_on_sc=True
    ),
)
def _bpr_kernel(*args):
    _bpr_body(*args)


PAD_BLOCK = 2000


def kernel(u, i, j, user_emb, item_emb):
    ue2 = user_emb.reshape(user_emb.shape[0] // 2, PAIR)
    ie2 = item_emb.reshape(item_emb.shape[0] // 2, PAIR)
    return _bpr_kernel(
        u.astype(jnp.int32), i.astype(jnp.int32), j.astype(jnp.int32),
        ue2, ie2,
    )


# speedup vs baseline: 1.0628x; 1.0628x over previous
import functools

import jax
import jax.numpy as jnp
from jax import lax
from jax.experimental import pallas as pl
from jax.experimental.pallas import tpu as pltpu
from jax.experimental.pallas import tpu_sc as plsc

BATCH = 16384
D = 64
PAIR = 2 * D
NC = 2
NS = 16
L = 16
NW = NC * NS
BPW = BATCH // NW
CHUNK = 128
NCHUNK = BPW // CHUNK
GPC = CHUNK // L


def _bpr_body(u_hbm, i_hbm, j_hbm, ue_hbm, ie_hbm, out_hbm,
              idx_u, idx_i, idx_j, pr_i, pr_j,
              rows_u, rows_i, rows_j, out_v, sem0, sem1):
    wid = lax.axis_index("s") * NC + lax.axis_index("c")
    base = wid * BPW

    pltpu.sync_copy(u_hbm.at[pl.ds(base, BPW)], idx_u)
    pltpu.sync_copy(i_hbm.at[pl.ds(base, BPW)], idx_i)
    pltpu.sync_copy(j_hbm.at[pl.ds(base, BPW)], idx_j)

    sems = (sem0, sem1)
    lanes = lax.iota(jnp.int32, L)

    def halve_chunk(c):
        def g_body(g, carry):
            o = c * CHUNK + g * L
            pr_i[pl.ds(o, L)] = lax.shift_right_logical(idx_i[pl.ds(o, L)], 1)
            pr_j[pl.ds(o, L)] = lax.shift_right_logical(idx_j[pl.ds(o, L)], 1)
            return carry
        lax.fori_loop(0, GPC, g_body, 0)

    def fire_chunk(c):
        buf = c % 2
        sl = pl.ds(c * CHUNK, CHUNK)
        return (
            pltpu.async_copy(ue_hbm.at[idx_u.at[sl]], rows_u.at[buf], sems[buf]),
            pltpu.async_copy(ie_hbm.at[pr_i.at[sl]], rows_i.at[buf], sems[buf]),
            pltpu.async_copy(ie_hbm.at[pr_j.at[sl]], rows_j.at[buf], sems[buf]),
        )

    def compute_chunk(c):
        buf = c % 2
        ru, ri, rj = rows_u.at[buf], rows_i.at[buf], rows_j.at[buf]

        def group_body(g, carry):
            o = c * CHUNK + g * L
            rows_in = g * L + lanes
            half_i = (idx_i[pl.ds(o, L)] & 1) * D
            half_j = (idx_j[pl.ds(o, L)] & 1) * D
            acc = jnp.zeros((L,), jnp.float32)
            for d in range(D):
                col = (lanes + d) & (D - 1)
                ue = plsc.load_gather(ru, [rows_in, col])
                ie = plsc.load_gather(ri, [rows_in, half_i + col])
                je = plsc.load_gather(rj, [rows_in, half_j + col])
                acc = acc + ue * (ie - je)
            out_v[pl.ds(o, L)] = acc
            return carry

        lax.fori_loop(0, GPC, group_body, 0)

    halve_chunk(0)
    copies = fire_chunk(0)
    for c in range(NCHUNK):
        if c + 1 < NCHUNK:
            halve_chunk(c + 1)
            next_copies = fire_chunk(c + 1)
        for cp in copies:
            cp.wait()
        compute_chunk(c)
        if c + 1 < NCHUNK:
            copies = next_copies

    pltpu.sync_copy(out_v, out_hbm.at[pl.ds(base, BPW)])


@functools.partial(
    pl.kernel,
    out_type=jax.ShapeDtypeStruct((BATCH,), jnp.float32),
    mesh=plsc.VectorSubcoreMesh(
        core_axis_name="c", subcore_axis_name="s", num_cores=NC, num_subcores=NS
    ),
    scratch_types=[
        pltpu.VMEM((BPW,), jnp.int32),
        pltpu.VMEM((BPW,), jnp.int32),
        pltpu.VMEM((BPW,), jnp.int32),
        pltpu.VMEM((BPW,), jnp.int32),
        pltpu.VMEM((BPW,), jnp.int32),
        pltpu.VMEM((2, CHUNK, PAIR), jnp.float32),
        pltpu.VMEM((2, CHUNK, PAIR), jnp.float32),
        pltpu.VMEM((2, CHUNK, PAIR), jnp.float32),
        pltpu.VMEM((BPW,), jnp.float32),
        pltpu.SemaphoreType.DMA,
        pltpu.SemaphoreType.DMA,
    ],
    compiler_params=pltpu.CompilerParams(
        needs_layout_passes=False, use_tc_tiling_on_sc=True
    ),
)
def _bpr_kernel(*args):
    _bpr_body(*args)


PAD_BLOCK = 2000


def kernel(u, i, j, user_emb, item_emb):
    ue_p = jnp.pad(user_emb, ((0, 0), (0, D)))
    ie2 = item_emb.reshape(item_emb.shape[0] // 2, PAIR)
    return _bpr_kernel(
        u.astype(jnp.int32), i.astype(jnp.int32), j.astype(jnp.int32),
        ue_p, ie2,
    )
